# baseline (speedup 1.0000x reference)
import functools

import numpy as np
import jax
import jax.numpy as jnp
from jax import lax
from jax.experimental import pallas as pl
from jax.experimental.pallas import tpu as pltpu
from jax.experimental.pallas import tpu_sc as plsc

_CUTOFF = 5.0
_LOG2 = float(np.log(2.0))
_PI = float(np.pi)


def _ssp(v):
    return jnp.maximum(v, 0.0) + jnp.log1p(jnp.exp(-jnp.abs(v))) - _LOG2


def _cutoff_body(r_ref, m_ref, c_ref):
    r = r_ref[...]
    c = 0.5 * (jnp.cos(r * (_PI / _CUTOFF)) + 1.0)
    c = jnp.where(r < _CUTOFF, c, 0.0) * m_ref[...]
    c_ref[...] = jnp.broadcast_to(c[:, None, :], (c.shape[0], 8, c.shape[1]))


def _in2f_body(x_ref, w_ref, y_ref):
    y_ref[...] = jnp.dot(
        x_ref[...], w_ref[...], preferred_element_type=jnp.float32
    ).astype(y_ref.dtype)


def _fused_body(f_ref, yg_ref, c_ref, w1_ref, b1_ref, w2_ref, b2_ref,
                wo_ref, bo_ref, o_ref):
    npair = f_ref.shape[1]
    nbh = f_ref.shape[2]
    blk = npair * nbh
    shift = int(np.log2(nbh))
    f2 = f_ref[...].reshape(blk, f_ref.shape[3])
    h = jnp.dot(f2, w1_ref[...], preferred_element_type=jnp.float32)
    h = _ssp(h + b1_ref[...])
    wf = jnp.dot(h, w2_ref[...], preferred_element_type=jnp.float32) + b2_ref[...]
    m = wf * yg_ref[...].astype(jnp.float32)
    pid = lax.broadcasted_iota(jnp.int32, (npair, blk), 0)
    eid = lax.broadcasted_iota(jnp.int32, (npair, blk), 1)
    sel = jnp.where((eid >> shift) == pid, c_ref[0, 0:1, :], 0.0)
    agg = jnp.dot(sel, m, preferred_element_type=jnp.float32)
    o_ref[...] = _ssp(
        jnp.dot(agg, wo_ref[...], preferred_element_type=jnp.float32) + bo_ref[...]
    )


@functools.lru_cache(maxsize=None)
def _make_sc_gather(nchunks, feat, dt):
    info = plsc.get_sparse_core_info()
    nc, ns = info.num_cores, info.num_subcores
    nw = nc * ns
    CH = 128
    nch = nchunks // nw
    NBUF = 4
    GSP = 32
    NGS = CH // GSP
    mesh = plsc.VectorSubcoreMesh(core_axis_name="c", subcore_axis_name="s")

    @functools.partial(
        pl.kernel,
        mesh=mesh,
        out_type=jax.ShapeDtypeStruct((nchunks * CH, feat), dt),
        scratch_types=[
            pltpu.VMEM((nch, CH), jnp.int32),
            pltpu.VMEM((NBUF, CH, feat), dt),
        ]
        + [pltpu.SemaphoreType.DMA] * (2 * NBUF),
    )
    def gather(y_hbm, idx_hbm, out_hbm, idx_v, rows_v, *sems):
        gs, os = sems[:NBUF], sems[NBUF:]
        w = lax.axis_index("s") * nc + lax.axis_index("c")
        pltpu.sync_copy(idx_hbm.at[pl.ds(w * nch, nch)], idx_v)
        e0 = w * nch * CH

        def g_issue(ci, bi):
            for i in range(NGS):
                pltpu.async_copy(
                    y_hbm.at[idx_v.at[ci, pl.ds(i * GSP, GSP)]],
                    rows_v.at[bi, pl.ds(i * GSP, GSP)],
                    gs[bi],
                )

        def g_wait(bi):
            for i in range(NGS):
                pltpu.make_async_copy(
                    y_hbm.at[pl.ds(0, GSP)],
                    rows_v.at[bi, pl.ds(i * GSP, GSP)],
                    gs[bi],
                ).wait()

        def o_issue(ci, bi):
            pltpu.async_copy(
                rows_v.at[bi], out_hbm.at[pl.ds(e0 + ci * CH, CH)], os[bi]
            )

        def o_wait(bi):
            pltpu.make_async_copy(
                rows_v.at[bi], out_hbm.at[pl.ds(0, CH)], os[bi]
            ).wait()

        for b in range(NBUF):
            g_issue(b, b)

        def sup(u, carry):
            c0 = u * NBUF
            for b in range(NBUF):
                g_wait(b)
                o_issue(c0 + b, b)
            for b in range(NBUF):
                @pl.when(u < nch // NBUF - 1)
                def _(b=b):
                    o_wait(b)
                    g_issue(c0 + NBUF + b, b)
            return carry

        lax.fori_loop(0, nch // NBUF, sup, 0)
        for b in range(NBUF):
            o_wait(b)

    return gather


def kernel(x, r_ij, neighbors, pairwise_mask, f_ij, W1, b1, W2, b2, Win, Wout, bout):
    B, N, F = x.shape
    NBH = neighbors.shape[2]
    NG = f_ij.shape[3]
    ROWS = B * N * NBH
    PAIRS = B * N

    NBLK = 32
    EBLK = NBLK * NBH
    Q = ROWS // EBLK
    rd = r_ij.reshape(Q, EBLK)
    md = pairwise_mask.reshape(Q, EBLK)

    CB = 8
    c3 = pl.pallas_call(
        _cutoff_body,
        grid=(Q // CB,),
        in_specs=[
            pl.BlockSpec((CB, EBLK), lambda i: (i, 0)),
            pl.BlockSpec((CB, EBLK), lambda i: (i, 0)),
        ],
        out_specs=pl.BlockSpec((CB, 8, EBLK), lambda i: (i, 0, 0)),
        out_shape=jax.ShapeDtypeStruct((Q, 8, EBLK), jnp.float32),
    )(rd, md)

    y2 = pl.pallas_call(
        _in2f_body,
        out_shape=jax.ShapeDtypeStruct((PAIRS, F), jnp.float32),
    )(x.reshape(PAIRS, F), Win)

    S = 4
    MS = B // S
    EPM = N * NBH
    nb32 = neighbors.astype(jnp.int32)
    gidx = (nb32 + (jnp.arange(B, dtype=jnp.int32) * N)[:, None, None]).reshape(
        S, MS * EPM // 128, 128
    )
    gat = _make_sc_gather(MS * EPM // 128, F, jnp.float32)

    outs = []
    for s in range(S):
        yg_s = gat(y2, gidx[s])
        out_s = pl.pallas_call(
            _fused_body,
            grid=(MS, N // NBLK),
            in_specs=[
                pl.BlockSpec(
                    (1, NBLK, NBH, NG), lambda b, j, s=s: (s * MS + b, j, 0, 0)
                ),
                pl.BlockSpec((EBLK, F), lambda b, j: (b * (N // NBLK) + j, 0)),
                pl.BlockSpec(
                    (1, 8, EBLK),
                    lambda b, j, s=s: ((s * MS + b) * (N // NBLK) + j, 0, 0),
                ),
                pl.BlockSpec((NG, F), lambda b, j: (0, 0)),
                pl.BlockSpec((1, F), lambda b, j: (0, 0)),
                pl.BlockSpec((F, F), lambda b, j: (0, 0)),
                pl.BlockSpec((1, F), lambda b, j: (0, 0)),
                pl.BlockSpec((F, F), lambda b, j: (0, 0)),
                pl.BlockSpec((1, F), lambda b, j: (0, 0)),
            ],
            out_specs=pl.BlockSpec(
                (NBLK, F), lambda b, j: (b * (N // NBLK) + j, 0)
            ),
            out_shape=jax.ShapeDtypeStruct((MS * N, F), jnp.float32),
        )(
            f_ij,
            yg_s,
            c3,
            W1,
            b1.reshape(1, F),
            W2,
            b2.reshape(1, F),
            Wout,
            bout.reshape(1, F),
        )
        outs.append(out_s)
    return jnp.concatenate(outs, axis=0).reshape(B, N, F)
